# baseline (speedup 1.0000x reference)
import functools

import jax
import jax.numpy as jnp
import numpy as np
from jax import lax
from jax.experimental import pallas as pl
from jax.experimental.pallas import tpu as pltpu
from jax.experimental.pallas import tpu_sc as plsc

N = 10000
NP = 10240
TRASH = N
E = 320000
CHUNK = 256
NTILES = 16
CPT = 80
EPT = CPT * CHUNK
EP = EPT * NTILES
EPC = EP // CHUNK
ROWS_PT = NP // NTILES
D = 128
H = 256
BN_SCALE = float(1.0 / np.sqrt(1.0 + 1e-5))
_PREC = lax.Precision.HIGHEST


def _make_sc_agg(Fh, with_count):
    mesh = plsc.VectorSubcoreMesh(core_axis_name="core", subcore_axis_name="subcore")
    out_type = [jax.ShapeDtypeStruct((NP, Fh), jnp.float32),
                jax.ShapeDtypeStruct((NP, Fh), jnp.float32)]
    scratch = [
        pltpu.VMEM((CHUNK,), jnp.int32),
        pltpu.VMEM((CHUNK,), jnp.int32),
        pltpu.VMEM((CHUNK, Fh), jnp.float32),
        pltpu.VMEM_SHARED((NP, Fh), jnp.float32),
    ]
    if with_count:
        out_type.append(jax.ShapeDtypeStruct((NP, 16), jnp.float32))
        scratch += [
            pltpu.VMEM((CHUNK, 16), jnp.float32),
            pltpu.VMEM_SHARED((NP, 16), jnp.float32),
        ]

    @functools.partial(pl.kernel, out_type=out_type, mesh=mesh,
                       scratch_types=scratch,
                       compiler_params=pltpu.CompilerParams(
                           use_tc_tiling_on_sc=False))
    def body(hlo, hhi, src2d, dst2d, out_lo, out_hi, *rest):
        if with_count:
            cnt_out, sidx, didx, rows, acc, ones, cacc = rest
        else:
            sidx, didx, rows, acc = rest
        c = lax.axis_index("core")
        t = lax.axis_index("subcore")
        tbase = t * CPT

        @pl.loop(0, CHUNK)
        def _(r):
            @pl.loop(0, Fh // 16)
            def _(j):
                rows.at[r, pl.ds(j * 16, 16)][...] = jnp.zeros((16,), jnp.float32)
            if with_count:
                ones.at[r, pl.ds(0, 16)][...] = jnp.zeros((16,), jnp.float32)

        for zoff, zn in ((0, 256), (256, 256), (512, 128)):
            zslc = pl.ds(t * ROWS_PT + zoff, zn)
            pltpu.sync_copy(rows.at[pl.ds(0, zn)], acc.at[zslc])
            if with_count:
                @pl.when(c == 0)
                def _():
                    pltpu.sync_copy(ones.at[pl.ds(0, zn)], cacc.at[zslc])

        if with_count:
            @pl.loop(0, CHUNK)
            def _(r):
                ones.at[r, pl.ds(0, 16)][...] = jnp.ones((16,), jnp.float32)

        plsc.subcore_barrier()

        @pl.loop(0, CPT)
        def _(g):
            rowb = tbase + g
            pltpu.sync_copy(src2d.at[rowb], sidx)
            pltpu.sync_copy(dst2d.at[rowb], didx)

            def do_group(h_hbm, do_count):
                pltpu.sync_copy(h_hbm.at[sidx], rows)
                pltpu.sync_copy(rows, acc.at[didx], add=True)
                if do_count:
                    pltpu.sync_copy(ones, cacc.at[didx], add=True)

            @pl.when(c == 0)
            def _():
                do_group(hlo, with_count)

            @pl.when(c == 1)
            def _():
                do_group(hhi, False)

        plsc.subcore_barrier()

        rs = pl.ds(t * ROWS_PT, ROWS_PT)

        @pl.when(c == 0)
        def _():
            pltpu.sync_copy(acc.at[rs], out_lo.at[rs])
            if with_count:
                pltpu.sync_copy(cacc.at[rs], cnt_out.at[rs])

        @pl.when(c == 1)
        def _():
            pltpu.sync_copy(acc.at[rs], out_hi.at[rs])

    return body


_sc_agg0 = _make_sc_agg(D // 2, with_count=True)
_sc_agg = _make_sc_agg(H // 2, with_count=False)


_R = 2048


def _dot_t(a, w):
    return lax.dot_general(a, w, (((1,), (1,)), ((), ())),
                           precision=_PREC, preferred_element_type=jnp.float32)


def _sage_block(ag, cnt, hh, wl, bl, wr, g, be):
    inv = 1.0 / jnp.maximum(cnt[...][:, 0:1], 1.0)
    agg = jnp.concatenate([ag[...][0], ag[...][1]], axis=1) * inv
    h = jnp.concatenate([hh[...][0], hh[...][1]], axis=1)
    z = _dot_t(agg, wl[...]) + _dot_t(h, wr[...]) + bl[...]
    return jnp.maximum(g[...] * (z * BN_SCALE) + be[...], 0.0)


def _layer_body(ag, cnt, hh, wl, bl, wr, g, be, olo, ohi):
    hn = _sage_block(ag, cnt, hh, wl, bl, wr, g, be)
    olo[...] = hn[:, : H // 2]
    ohi[...] = hn[:, H // 2:]


def _head_body(ag, cnt, hh, wl, bl, wr, g, be, wc1, bc1, wc2, bc2, out):
    h3 = _sage_block(ag, cnt, hh, wl, bl, wr, g, be)
    t1 = jnp.maximum(_dot_t(h3, wc1[...]) + bc1[...], 0.0)
    logits = _dot_t(t1, wc2[...]) + bc2[...]
    m = jnp.max(logits, axis=1, keepdims=True)
    lse = m + jnp.log(jnp.sum(jnp.exp(logits - m), axis=1, keepdims=True))
    out[...] = logits - lse


def _stk_spec(fw):
    return pl.BlockSpec((2, _R, fw), lambda i: (0, i, 0))


def _row_spec(fw):
    return pl.BlockSpec((_R, fw), lambda i: (i, 0))


def _full_spec(shape):
    return pl.BlockSpec(shape, lambda i: tuple(0 for _ in shape))


def _make_tc_layer(Fin):
    in_specs = [
        _stk_spec(Fin // 2), _row_spec(16), _stk_spec(Fin // 2),
        _full_spec((H, Fin)), _full_spec((1, H)),
        _full_spec((H, Fin)),
        _full_spec((1, H)), _full_spec((1, H)),
    ]
    return pl.pallas_call(
        _layer_body,
        grid=(NP // _R,),
        in_specs=in_specs,
        out_specs=[_row_spec(H // 2), _row_spec(H // 2)],
        out_shape=[jax.ShapeDtypeStruct((NP, H // 2), jnp.float32)] * 2,
    )


_tc_layer0 = _make_tc_layer(D)
_tc_layer = _make_tc_layer(H)

_tc_head = pl.pallas_call(
    _head_body,
    grid=(NP // _R,),
    in_specs=[
        _stk_spec(H // 2), _row_spec(16), _stk_spec(H // 2),
        _full_spec((H, H)), _full_spec((1, H)),
        _full_spec((H, H)),
        _full_spec((1, H)), _full_spec((1, H)),
        _full_spec((H // 2, H)), _full_spec((1, H // 2)),
        _full_spec((2, H // 2)), _full_spec((1, 2)),
    ],
    out_specs=[_row_spec(2)],
    out_shape=[jax.ShapeDtypeStruct((NP, 2), jnp.float32)],
)


def kernel(x, edge_index, Wl0, bl0, Wr0, g0, be0, Wl1, bl1, Wr1, g1, be1,
           Wl2, bl2, Wr2, g2, be2, Wc1, bc1, Wc2, bc2):
    src = edge_index[0]
    dst = edge_index[1]
    pad = EP - E
    src2d = jnp.concatenate([src, jnp.zeros((pad,), jnp.int32)]).reshape(EPC, CHUNK)
    dst2d = jnp.concatenate([dst, jnp.full((pad,), TRASH, jnp.int32)]).reshape(EPC, CHUNK)

    xp = jnp.pad(x, ((0, NP - N), (0, 0)))
    xs = jnp.stack([xp[:, : D // 2], xp[:, D // 2:]], axis=0)

    r = lambda v: v.reshape(1, -1)

    x_lo = xp[:, : D // 2]
    x_hi = xp[:, D // 2:]
    agg0_lo, agg0_hi, cnt = _sc_agg0(x_lo, x_hi, src2d, dst2d)
    agg0 = jnp.stack([agg0_lo, agg0_hi], axis=0)
    h1_lo, h1_hi = _tc_layer0(agg0, cnt, xs, Wl0, r(bl0), Wr0, r(g0), r(be0))
    agg1_lo, agg1_hi = _sc_agg(h1_lo, h1_hi, src2d, dst2d)
    agg1 = jnp.stack([agg1_lo, agg1_hi], axis=0)
    h1 = jnp.stack([h1_lo, h1_hi], axis=0)
    h2_lo, h2_hi = _tc_layer(agg1, cnt, h1, Wl1, r(bl1), Wr1, r(g1), r(be1))
    agg2_lo, agg2_hi = _sc_agg(h2_lo, h2_hi, src2d, dst2d)
    agg2 = jnp.stack([agg2_lo, agg2_hi], axis=0)
    h2 = jnp.stack([h2_lo, h2_hi], axis=0)
    (out_p,) = _tc_head(agg2, cnt, h2, Wl2, r(bl2), Wr2, r(g2), r(be2),
                        Wc1, r(bc1), Wc2, r(bc2))
    return out_p[:N]


# speedup vs baseline: 1.4145x; 1.4145x over previous
import functools

import jax
import jax.numpy as jnp
import numpy as np
from jax import lax
from jax.experimental import pallas as pl
from jax.experimental.pallas import tpu as pltpu
from jax.experimental.pallas import tpu_sc as plsc

N = 10000
NP = 10240
TRASH = N
E = 320000
CHUNK = 128
NTILES = 16
NCHUNKS = 157
EPT = CHUNK * NCHUNKS
EP = EPT * NTILES
ROWS_PT = NP // NTILES
D = 128
H = 256
BN_SCALE = float(1.0 / np.sqrt(1.0 + 1e-5))
_PREC = lax.Precision.HIGHEST


def _make_sc_agg(Fh, with_count):
    mesh = plsc.VectorSubcoreMesh(core_axis_name="core", subcore_axis_name="subcore")
    out_type = [
        jax.ShapeDtypeStruct((NP, Fh), jnp.float32),
        jax.ShapeDtypeStruct((NP, Fh), jnp.float32),
    ]
    scratch = [
        pltpu.VMEM((CHUNK,), jnp.int32),
        pltpu.VMEM((CHUNK,), jnp.int32),
        pltpu.VMEM((CHUNK, Fh), jnp.float32),
        pltpu.VMEM((CHUNK, Fh), jnp.float32),
        pltpu.VMEM_SHARED((NP, Fh), jnp.float32),
    ]
    if with_count:
        out_type.append(jax.ShapeDtypeStruct((NP, 16), jnp.float32))
        scratch += [
            pltpu.VMEM((CHUNK, 16), jnp.float32),
            pltpu.VMEM_SHARED((NP, 16), jnp.float32),
        ]

    @functools.partial(pl.kernel, out_type=out_type, mesh=mesh,
                       scratch_types=scratch,
                       compiler_params=pltpu.CompilerParams(
                           use_tc_tiling_on_sc=False))
    def body(hlo, hhi, src, dst, out_lo, out_hi, *rest):
        if with_count:
            cnt_out, sidx, didx, rows, zb, acc, ones, cacc = rest
        else:
            sidx, didx, rows, zb, acc = rest
        c = lax.axis_index("core")
        t = lax.axis_index("subcore")

        @pl.loop(0, CHUNK)
        def _(r):
            @pl.loop(0, Fh // 16)
            def _(j):
                zb.at[r, pl.ds(j * 16, 16)][...] = jnp.zeros((16,), jnp.float32)
            if with_count:
                ones.at[r, pl.ds(0, 16)][...] = jnp.zeros((16,), jnp.float32)

        @pl.loop(0, ROWS_PT // CHUNK)
        def _(j):
            pltpu.sync_copy(zb, acc.at[pl.ds(t * ROWS_PT + j * CHUNK, CHUNK)])
            if with_count:
                @pl.when(c == 0)
                def _():
                    pltpu.sync_copy(ones, cacc.at[pl.ds(t * ROWS_PT + j * CHUNK, CHUNK)])

        if with_count:
            @pl.loop(0, CHUNK)
            def _(r):
                ones.at[r, pl.ds(0, 16)][...] = jnp.ones((16,), jnp.float32)

        plsc.subcore_barrier()

        base = t * EPT

        @pl.loop(0, NCHUNKS)
        def _(i):
            off = base + i * CHUNK
            pltpu.sync_copy(src.at[pl.ds(off, CHUNK)], sidx)
            pltpu.sync_copy(dst.at[pl.ds(off, CHUNK)], didx)

            @pl.when(c == 0)
            def _():
                pltpu.sync_copy(hlo.at[sidx], rows)

            @pl.when(c == 1)
            def _():
                pltpu.sync_copy(hhi.at[sidx], rows)

            pltpu.sync_copy(rows, acc.at[didx], add=True)
            if with_count:
                @pl.when(c == 0)
                def _():
                    pltpu.sync_copy(ones, cacc.at[didx], add=True)

        plsc.subcore_barrier()

        rs = pl.ds(t * ROWS_PT, ROWS_PT)

        @pl.when(c == 0)
        def _():
            pltpu.sync_copy(acc.at[rs], out_lo.at[rs])
            if with_count:
                pltpu.sync_copy(cacc.at[rs], cnt_out.at[rs])

        @pl.when(c == 1)
        def _():
            pltpu.sync_copy(acc.at[rs], out_hi.at[rs])

    return body


_sc_agg0 = _make_sc_agg(D // 2, with_count=True)
_sc_agg = _make_sc_agg(H // 2, with_count=False)


_R = 2048


def _dot_t(a, w):
    return lax.dot_general(a, w, (((1,), (1,)), ((), ())),
                           precision=_PREC, preferred_element_type=jnp.float32)


def _layer_body(agl, agh, cnt, hl, hh, wl, bl, wr, g, be, olo, ohi):
    inv = 1.0 / jnp.maximum(cnt[...][:, 0:1], 1.0)
    agg = jnp.concatenate([agl[...], agh[...]], axis=1) * inv
    h = jnp.concatenate([hl[...], hh[...]], axis=1)
    z = _dot_t(agg, wl[...]) + _dot_t(h, wr[...]) + bl[...]
    hn = jnp.maximum(g[...] * (z * BN_SCALE) + be[...], 0.0)
    olo[...] = hn[:, : H // 2]
    ohi[...] = hn[:, H // 2:]


def _head_body(agl, agh, cnt, hl, hh, wl, bl, wr, g, be, wc1, bc1, wc2, bc2, out):
    inv = 1.0 / jnp.maximum(cnt[...][:, 0:1], 1.0)
    agg = jnp.concatenate([agl[...], agh[...]], axis=1) * inv
    h = jnp.concatenate([hl[...], hh[...]], axis=1)
    z = _dot_t(agg, wl[...]) + _dot_t(h, wr[...]) + bl[...]
    h3 = jnp.maximum(g[...] * (z * BN_SCALE) + be[...], 0.0)
    t1 = jnp.maximum(_dot_t(h3, wc1[...]) + bc1[...], 0.0)
    logits = _dot_t(t1, wc2[...]) + bc2[...]
    m = jnp.max(logits, axis=1, keepdims=True)
    lse = m + jnp.log(jnp.sum(jnp.exp(logits - m), axis=1, keepdims=True))
    out[...] = logits - lse


def _row_spec(fw):
    return pl.BlockSpec((_R, fw), lambda i: (i, 0))


def _full_spec(shape):
    return pl.BlockSpec(shape, lambda i: tuple(0 for _ in shape))


def _make_tc_layer(Fin):
    Fh = Fin // 2
    in_specs = [
        _row_spec(Fh), _row_spec(Fh), _row_spec(16),
        _row_spec(Fh), _row_spec(Fh),
        _full_spec((H, Fin)), _full_spec((1, H)),
        _full_spec((H, Fin)),
        _full_spec((1, H)), _full_spec((1, H)),
    ]
    return pl.pallas_call(
        _layer_body,
        grid=(NP // _R,),
        in_specs=in_specs,
        out_specs=[_row_spec(H // 2), _row_spec(H // 2)],
        out_shape=[jax.ShapeDtypeStruct((NP, H // 2), jnp.float32)] * 2,
    )


_tc_layer0 = _make_tc_layer(D)
_tc_layer = _make_tc_layer(H)

_tc_head = pl.pallas_call(
    _head_body,
    grid=(NP // _R,),
    in_specs=[
        _row_spec(H // 2), _row_spec(H // 2), _row_spec(16),
        _row_spec(H // 2), _row_spec(H // 2),
        _full_spec((H, H)), _full_spec((1, H)),
        _full_spec((H, H)),
        _full_spec((1, H)), _full_spec((1, H)),
        _full_spec((H // 2, H)), _full_spec((1, H // 2)),
        _full_spec((2, H // 2)), _full_spec((1, 2)),
    ],
    out_specs=[_row_spec(2)],
    out_shape=[jax.ShapeDtypeStruct((NP, 2), jnp.float32)],
)


def kernel(x, edge_index, Wl0, bl0, Wr0, g0, be0, Wl1, bl1, Wr1, g1, be1,
           Wl2, bl2, Wr2, g2, be2, Wc1, bc1, Wc2, bc2):
    src = edge_index[0]
    dst = edge_index[1]
    pad = EP - E
    src_p = jnp.concatenate([src, jnp.zeros((pad,), jnp.int32)])
    dst_p = jnp.concatenate([dst, jnp.full((pad,), TRASH, jnp.int32)])

    xp = jnp.pad(x, ((0, NP - N), (0, 0)))
    x_lo = xp[:, : D // 2]
    x_hi = xp[:, D // 2:]

    r = lambda v: v.reshape(1, -1)

    agg0_lo, agg0_hi, cnt = _sc_agg0(x_lo, x_hi, src_p, dst_p)
    h1_lo, h1_hi = _tc_layer0(agg0_lo, agg0_hi, cnt, x_lo, x_hi,
                              Wl0, r(bl0), Wr0, r(g0), r(be0))
    agg1_lo, agg1_hi = _sc_agg(h1_lo, h1_hi, src_p, dst_p)
    h2_lo, h2_hi = _tc_layer(agg1_lo, agg1_hi, cnt, h1_lo, h1_hi,
                             Wl1, r(bl1), Wr1, r(g1), r(be1))
    agg2_lo, agg2_hi = _sc_agg(h2_lo, h2_hi, src_p, dst_p)
    (out_p,) = _tc_head(agg2_lo, agg2_hi, cnt, h2_lo, h2_hi,
                        Wl2, r(bl2), Wr2, r(g2), r(be2),
                        Wc1, r(bc1), Wc2, r(bc2))
    return out_p[:N]


# speedup vs baseline: 1.8774x; 1.3273x over previous
import functools

import jax
import jax.numpy as jnp
import numpy as np
from jax import lax
from jax.experimental import pallas as pl
from jax.experimental.pallas import tpu as pltpu
from jax.experimental.pallas import tpu_sc as plsc

N = 10000
NP = 10240
TRASH = N
E = 320000
CHUNK = 128
NTILES = 16
NCHUNKS = 157
EPT = CHUNK * NCHUNKS
EP = EPT * NTILES
ROWS_PT = NP // NTILES
D = 128
H = 256
BN_SCALE = float(1.0 / np.sqrt(1.0 + 1e-5))
_PREC = lax.Precision.HIGHEST


def _make_sc_agg(Fh, with_count):
    mesh = plsc.VectorSubcoreMesh(core_axis_name="core", subcore_axis_name="subcore")
    out_type = [
        jax.ShapeDtypeStruct((NP, Fh), jnp.float32),
        jax.ShapeDtypeStruct((NP, Fh), jnp.float32),
    ]
    scratch = [
        pltpu.VMEM((CHUNK,), jnp.int32),
        pltpu.VMEM((CHUNK,), jnp.int32),
        pltpu.VMEM((CHUNK,), jnp.int32),
        pltpu.VMEM((CHUNK,), jnp.int32),
        pltpu.VMEM((CHUNK, Fh), jnp.float32),
        pltpu.VMEM((CHUNK, Fh), jnp.float32),
        pltpu.VMEM_SHARED((NP, Fh), jnp.float32),
        pltpu.SemaphoreType.DMA,
        pltpu.SemaphoreType.DMA,
    ]
    if with_count:
        out_type.append(jax.ShapeDtypeStruct((NP, 16), jnp.float32))
        scratch += [
            pltpu.VMEM((CHUNK, 16), jnp.float32),
            pltpu.VMEM_SHARED((NP, 16), jnp.float32),
        ]

    @functools.partial(pl.kernel, out_type=out_type, mesh=mesh,
                       scratch_types=scratch,
                       compiler_params=pltpu.CompilerParams(
                           use_tc_tiling_on_sc=False))
    def body(hlo, hhi, src, dst, out_lo, out_hi, *rest):
        if with_count:
            (cnt_out, sidxA, didxA, sidxB, didxB, rows, zb, acc,
             semA, semB, ones, cacc) = rest
        else:
            sidxA, didxA, sidxB, didxB, rows, zb, acc, semA, semB = rest
        c = lax.axis_index("core")
        t = lax.axis_index("subcore")

        @pl.loop(0, CHUNK)
        def _(r):
            @pl.loop(0, Fh // 16)
            def _(j):
                zb.at[r, pl.ds(j * 16, 16)][...] = jnp.zeros((16,), jnp.float32)
            if with_count:
                ones.at[r, pl.ds(0, 16)][...] = jnp.zeros((16,), jnp.float32)

        @pl.loop(0, ROWS_PT // CHUNK)
        def _(j):
            pltpu.sync_copy(zb, acc.at[pl.ds(t * ROWS_PT + j * CHUNK, CHUNK)])
            if with_count:
                @pl.when(c == 0)
                def _():
                    pltpu.sync_copy(ones, cacc.at[pl.ds(t * ROWS_PT + j * CHUNK, CHUNK)])

        if with_count:
            @pl.loop(0, CHUNK)
            def _(r):
                ones.at[r, pl.ds(0, 16)][...] = jnp.ones((16,), jnp.float32)

        plsc.subcore_barrier()

        base = t * EPT
        half = (NCHUNKS - 1) // 2

        def fetch(buf_s, buf_d, off, sem):
            pltpu.async_copy(src.at[pl.ds(off, CHUNK)], buf_s, sem)
            pltpu.async_copy(dst.at[pl.ds(off, CHUNK)], buf_d, sem)

        def wait_fetch(buf_s, buf_d, off, sem):
            pltpu.make_async_copy(src.at[pl.ds(off, CHUNK)], buf_s, sem).wait()
            pltpu.make_async_copy(dst.at[pl.ds(off, CHUNK)], buf_d, sem).wait()

        def process(buf_s, buf_d):
            @pl.when(c == 0)
            def _():
                pltpu.sync_copy(hlo.at[buf_s], rows)

            @pl.when(c == 1)
            def _():
                pltpu.sync_copy(hhi.at[buf_s], rows)

            pltpu.sync_copy(rows, acc.at[buf_d], add=True)
            if with_count:
                @pl.when(c == 0)
                def _():
                    pltpu.sync_copy(ones, cacc.at[buf_d], add=True)

        fetch(sidxA, didxA, base, semA)
        fetch(sidxB, didxB, base + CHUNK, semB)

        @pl.loop(0, half)
        def _(k):
            offa = base + (2 * k) * CHUNK
            wait_fetch(sidxA, didxA, offa, semA)
            process(sidxA, didxA)
            fetch(sidxA, didxA, offa + 2 * CHUNK, semA)

            offb = offa + CHUNK
            wait_fetch(sidxB, didxB, offb, semB)
            process(sidxB, didxB)

            @pl.when(k < half - 1)
            def _():
                fetch(sidxB, didxB, offb + 2 * CHUNK, semB)

        offz = base + (NCHUNKS - 1) * CHUNK
        wait_fetch(sidxA, didxA, offz, semA)
        process(sidxA, didxA)

        plsc.subcore_barrier()

        rs = pl.ds(t * ROWS_PT, ROWS_PT)

        @pl.when(c == 0)
        def _():
            pltpu.sync_copy(acc.at[rs], out_lo.at[rs])
            if with_count:
                pltpu.sync_copy(cacc.at[rs], cnt_out.at[rs])

        @pl.when(c == 1)
        def _():
            pltpu.sync_copy(acc.at[rs], out_hi.at[rs])

    return body


_sc_agg0 = _make_sc_agg(D // 2, with_count=True)
_sc_agg = _make_sc_agg(H // 2, with_count=False)


_R = 2048


def _dot_t(a, w):
    return lax.dot_general(a, w, (((1,), (1,)), ((), ())),
                           precision=_PREC, preferred_element_type=jnp.float32)


def _layer_body(agl, agh, cnt, hl, hh, wl, bl, wr, g, be, olo, ohi):
    inv = 1.0 / jnp.maximum(cnt[...][:, 0:1], 1.0)
    agg = jnp.concatenate([agl[...], agh[...]], axis=1) * inv
    h = jnp.concatenate([hl[...], hh[...]], axis=1)
    z = _dot_t(agg, wl[...]) + _dot_t(h, wr[...]) + bl[...]
    hn = jnp.maximum(g[...] * (z * BN_SCALE) + be[...], 0.0)
    olo[...] = hn[:, : H // 2]
    ohi[...] = hn[:, H // 2:]


def _head_body(agl, agh, cnt, hl, hh, wl, bl, wr, g, be, wc1, bc1, wc2, bc2, out):
    inv = 1.0 / jnp.maximum(cnt[...][:, 0:1], 1.0)
    agg = jnp.concatenate([agl[...], agh[...]], axis=1) * inv
    h = jnp.concatenate([hl[...], hh[...]], axis=1)
    z = _dot_t(agg, wl[...]) + _dot_t(h, wr[...]) + bl[...]
    h3 = jnp.maximum(g[...] * (z * BN_SCALE) + be[...], 0.0)
    t1 = jnp.maximum(_dot_t(h3, wc1[...]) + bc1[...], 0.0)
    logits = _dot_t(t1, wc2[...]) + bc2[...]
    m = jnp.max(logits, axis=1, keepdims=True)
    lse = m + jnp.log(jnp.sum(jnp.exp(logits - m), axis=1, keepdims=True))
    out[...] = logits - lse


def _row_spec(fw):
    return pl.BlockSpec((_R, fw), lambda i: (i, 0))


def _full_spec(shape):
    return pl.BlockSpec(shape, lambda i: tuple(0 for _ in shape))


def _make_tc_layer(Fin):
    Fh = Fin // 2
    in_specs = [
        _row_spec(Fh), _row_spec(Fh), _row_spec(16),
        _row_spec(Fh), _row_spec(Fh),
        _full_spec((H, Fin)), _full_spec((1, H)),
        _full_spec((H, Fin)),
        _full_spec((1, H)), _full_spec((1, H)),
    ]
    return pl.pallas_call(
        _layer_body,
        grid=(NP // _R,),
        in_specs=in_specs,
        out_specs=[_row_spec(H // 2), _row_spec(H // 2)],
        out_shape=[jax.ShapeDtypeStruct((NP, H // 2), jnp.float32)] * 2,
    )


_tc_layer0 = _make_tc_layer(D)
_tc_layer = _make_tc_layer(H)

_tc_head = pl.pallas_call(
    _head_body,
    grid=(NP // _R,),
    in_specs=[
        _row_spec(H // 2), _row_spec(H // 2), _row_spec(16),
        _row_spec(H // 2), _row_spec(H // 2),
        _full_spec((H, H)), _full_spec((1, H)),
        _full_spec((H, H)),
        _full_spec((1, H)), _full_spec((1, H)),
        _full_spec((H // 2, H)), _full_spec((1, H // 2)),
        _full_spec((2, H // 2)), _full_spec((1, 2)),
    ],
    out_specs=[_row_spec(2)],
    out_shape=[jax.ShapeDtypeStruct((NP, 2), jnp.float32)],
)


def kernel(x, edge_index, Wl0, bl0, Wr0, g0, be0, Wl1, bl1, Wr1, g1, be1,
           Wl2, bl2, Wr2, g2, be2, Wc1, bc1, Wc2, bc2):
    src = edge_index[0]
    dst = edge_index[1]
    pad = EP - E
    src_p = jnp.concatenate([src, jnp.zeros((pad,), jnp.int32)])
    dst_p = jnp.concatenate([dst, jnp.full((pad,), TRASH, jnp.int32)])

    xp = jnp.pad(x, ((0, NP - N), (0, 0)))
    x_lo = xp[:, : D // 2]
    x_hi = xp[:, D // 2:]

    r = lambda v: v.reshape(1, -1)

    agg0_lo, agg0_hi, cnt = _sc_agg0(x_lo, x_hi, src_p, dst_p)
    h1_lo, h1_hi = _tc_layer0(agg0_lo, agg0_hi, cnt, x_lo, x_hi,
                              Wl0, r(bl0), Wr0, r(g0), r(be0))
    agg1_lo, agg1_hi = _sc_agg(h1_lo, h1_hi, src_p, dst_p)
    h2_lo, h2_hi = _tc_layer(agg1_lo, agg1_hi, cnt, h1_lo, h1_hi,
                             Wl1, r(bl1), Wr1, r(g1), r(be1))
    agg2_lo, agg2_hi = _sc_agg(h2_lo, h2_hi, src_p, dst_p)
    (out_p,) = _tc_head(agg2_lo, agg2_hi, cnt, h2_lo, h2_hi,
                        Wl2, r(bl2), Wr2, r(g2), r(be2),
                        Wc1, r(bc1), Wc2, r(bc2))
    return out_p[:N]


# speedup vs baseline: 2.2752x; 1.2119x over previous
import functools

import jax
import jax.numpy as jnp
import numpy as np
from jax import lax
from jax.experimental import pallas as pl
from jax.experimental.pallas import tpu as pltpu
from jax.experimental.pallas import tpu_sc as plsc

N = 10000
NP = 10240
TRASH = N
E = 320000
CHUNK = 128
NTILES = 16
NCHUNKS = 157
EPT = CHUNK * NCHUNKS
EP = EPT * NTILES
ROWS_PT = NP // NTILES
D = 128
H = 256
BN_SCALE = float(1.0 / np.sqrt(1.0 + 1e-5))
_PREC = lax.Precision.HIGHEST


def _make_sc_agg(Fh, with_count):
    mesh = plsc.VectorSubcoreMesh(core_axis_name="core", subcore_axis_name="subcore")
    out_type = [
        jax.ShapeDtypeStruct((NP, Fh), jnp.float32),
        jax.ShapeDtypeStruct((NP, Fh), jnp.float32),
    ]
    scratch = [
        pltpu.VMEM((CHUNK,), jnp.int32),
        pltpu.VMEM((CHUNK,), jnp.int32),
        pltpu.VMEM((CHUNK,), jnp.int32),
        pltpu.VMEM((CHUNK,), jnp.int32),
        pltpu.VMEM((CHUNK,), jnp.int32),
        pltpu.VMEM((CHUNK,), jnp.int32),
        pltpu.VMEM((CHUNK, Fh), jnp.float32),
        pltpu.VMEM((CHUNK, Fh), jnp.float32),
        pltpu.VMEM_SHARED((NP, Fh), jnp.float32),
        pltpu.SemaphoreType.DMA,
        pltpu.SemaphoreType.DMA,
        pltpu.SemaphoreType.DMA,
        pltpu.SemaphoreType.DMA,
    ]
    if with_count:
        out_type.append(jax.ShapeDtypeStruct((NP, 16), jnp.float32))
        scratch += [
            pltpu.VMEM((CHUNK, 16), jnp.float32),
            pltpu.VMEM_SHARED((NP, 16), jnp.float32),
        ]

    @functools.partial(pl.kernel, out_type=out_type, mesh=mesh,
                       scratch_types=scratch,
                       compiler_params=pltpu.CompilerParams(
                           use_tc_tiling_on_sc=False))
    def body(hlo, hhi, src, dst, out_lo, out_hi, *rest):
        if with_count:
            (cnt_out, sidxA, didxA, sidxB, didxB, sdidxA, sdidxB,
             rowsA, rowsB, acc, semA, semB, semSA, semSB, ones, cacc) = rest
        else:
            (sidxA, didxA, sidxB, didxB, sdidxA, sdidxB,
             rowsA, rowsB, acc, semA, semB, semSA, semSB) = rest
        c = lax.axis_index("core")
        t = lax.axis_index("subcore")

        @pl.loop(0, CHUNK)
        def _(r):
            @pl.loop(0, Fh // 16)
            def _(j):
                rowsA.at[r, pl.ds(j * 16, 16)][...] = jnp.zeros((16,), jnp.float32)
            if with_count:
                ones.at[r, pl.ds(0, 16)][...] = jnp.zeros((16,), jnp.float32)

        @pl.loop(0, ROWS_PT // CHUNK)
        def _(j):
            pltpu.sync_copy(rowsA, acc.at[pl.ds(t * ROWS_PT + j * CHUNK, CHUNK)])
            if with_count:
                @pl.when(c == 0)
                def _():
                    pltpu.sync_copy(ones, cacc.at[pl.ds(t * ROWS_PT + j * CHUNK, CHUNK)])

        if with_count:
            @pl.loop(0, CHUNK)
            def _(r):
                ones.at[r, pl.ds(0, 16)][...] = jnp.ones((16,), jnp.float32)

        plsc.subcore_barrier()

        base = t * EPT
        half = (NCHUNKS - 1) // 2

        def fetch(buf_s, buf_d, off, sem):
            pltpu.async_copy(src.at[pl.ds(off, CHUNK)], buf_s, sem)
            pltpu.async_copy(dst.at[pl.ds(off, CHUNK)], buf_d, sem)

        def wait_fetch(buf_s, buf_d, off, sem):
            pltpu.make_async_copy(src.at[pl.ds(off, CHUNK)], buf_s, sem).wait()
            pltpu.make_async_copy(dst.at[pl.ds(off, CHUNK)], buf_d, sem).wait()

        def gather(buf_s, rows_):
            @pl.when(c == 0)
            def _():
                pltpu.sync_copy(hlo.at[buf_s], rows_)

            @pl.when(c == 1)
            def _():
                pltpu.sync_copy(hhi.at[buf_s], rows_)

        def snap_idx(buf_d, sbuf_d):
            for j in range(CHUNK // 16):
                sbuf_d.at[pl.ds(j * 16, 16)][...] = buf_d.at[pl.ds(j * 16, 16)][...]

        def wait_scatter(rows_, sbuf_d, semS_):
            pltpu.make_async_copy(rows_, acc.at[sbuf_d], semS_).wait()

        def phase(sidx_, didx_, sdidx_, rows_, semi_, semS_, off, prev_cond):
            wait_fetch(sidx_, didx_, off, semi_)
            if prev_cond is None:
                wait_scatter(rows_, sdidx_, semS_)
            else:
                @pl.when(prev_cond)
                def _():
                    wait_scatter(rows_, sdidx_, semS_)
            gather(sidx_, rows_)
            snap_idx(didx_, sdidx_)
            pltpu.async_copy(rows_, acc.at[sdidx_], semS_, add=True)
            if with_count:
                @pl.when(c == 0)
                def _():
                    pltpu.sync_copy(ones, cacc.at[sdidx_], add=True)

        fetch(sidxA, didxA, base, semA)
        fetch(sidxB, didxB, base + CHUNK, semB)

        @pl.loop(0, half)
        def _(k):
            offa = base + (2 * k) * CHUNK
            phase(sidxA, didxA, sdidxA, rowsA, semA, semSA, offa, k > 0)
            fetch(sidxA, didxA, offa + 2 * CHUNK, semA)

            offb = offa + CHUNK
            phase(sidxB, didxB, sdidxB, rowsB, semB, semSB, offb, k > 0)

            @pl.when(k < half - 1)
            def _():
                fetch(sidxB, didxB, offb + 2 * CHUNK, semB)

        offz = base + (NCHUNKS - 1) * CHUNK
        phase(sidxA, didxA, sdidxA, rowsA, semA, semSA, offz, None)
        wait_scatter(rowsA, sdidxA, semSA)
        wait_scatter(rowsB, sdidxB, semSB)

        plsc.subcore_barrier()

        rs = pl.ds(t * ROWS_PT, ROWS_PT)

        @pl.when(c == 0)
        def _():
            pltpu.sync_copy(acc.at[rs], out_lo.at[rs])
            if with_count:
                pltpu.sync_copy(cacc.at[rs], cnt_out.at[rs])

        @pl.when(c == 1)
        def _():
            pltpu.sync_copy(acc.at[rs], out_hi.at[rs])

    return body


_sc_agg0 = _make_sc_agg(D // 2, with_count=True)
_sc_agg = _make_sc_agg(H // 2, with_count=False)


_R = 2048


def _dot_t(a, w):
    return lax.dot_general(a, w, (((1,), (1,)), ((), ())),
                           precision=_PREC, preferred_element_type=jnp.float32)


def _layer_body(agl, agh, cnt, hl, hh, wl, bl, wr, g, be, olo, ohi):
    inv = 1.0 / jnp.maximum(cnt[...][:, 0:1], 1.0)
    agg = jnp.concatenate([agl[...], agh[...]], axis=1) * inv
    h = jnp.concatenate([hl[...], hh[...]], axis=1)
    z = _dot_t(agg, wl[...]) + _dot_t(h, wr[...]) + bl[...]
    hn = jnp.maximum(g[...] * (z * BN_SCALE) + be[...], 0.0)
    olo[...] = hn[:, : H // 2]
    ohi[...] = hn[:, H // 2:]


def _head_body(agl, agh, cnt, hl, hh, wl, bl, wr, g, be, wc1, bc1, wc2, bc2, out):
    inv = 1.0 / jnp.maximum(cnt[...][:, 0:1], 1.0)
    agg = jnp.concatenate([agl[...], agh[...]], axis=1) * inv
    h = jnp.concatenate([hl[...], hh[...]], axis=1)
    z = _dot_t(agg, wl[...]) + _dot_t(h, wr[...]) + bl[...]
    h3 = jnp.maximum(g[...] * (z * BN_SCALE) + be[...], 0.0)
    t1 = jnp.maximum(_dot_t(h3, wc1[...]) + bc1[...], 0.0)
    logits = _dot_t(t1, wc2[...]) + bc2[...]
    m = jnp.max(logits, axis=1, keepdims=True)
    lse = m + jnp.log(jnp.sum(jnp.exp(logits - m), axis=1, keepdims=True))
    out[...] = logits - lse


def _row_spec(fw):
    return pl.BlockSpec((_R, fw), lambda i: (i, 0))


def _full_spec(shape):
    return pl.BlockSpec(shape, lambda i: tuple(0 for _ in shape))


def _make_tc_layer(Fin):
    Fh = Fin // 2
    in_specs = [
        _row_spec(Fh), _row_spec(Fh), _row_spec(16),
        _row_spec(Fh), _row_spec(Fh),
        _full_spec((H, Fin)), _full_spec((1, H)),
        _full_spec((H, Fin)),
        _full_spec((1, H)), _full_spec((1, H)),
    ]
    return pl.pallas_call(
        _layer_body,
        grid=(NP // _R,),
        in_specs=in_specs,
        out_specs=[_row_spec(H // 2), _row_spec(H // 2)],
        out_shape=[jax.ShapeDtypeStruct((NP, H // 2), jnp.float32)] * 2,
    )


_tc_layer0 = _make_tc_layer(D)
_tc_layer = _make_tc_layer(H)

_tc_head = pl.pallas_call(
    _head_body,
    grid=(NP // _R,),
    in_specs=[
        _row_spec(H // 2), _row_spec(H // 2), _row_spec(16),
        _row_spec(H // 2), _row_spec(H // 2),
        _full_spec((H, H)), _full_spec((1, H)),
        _full_spec((H, H)),
        _full_spec((1, H)), _full_spec((1, H)),
        _full_spec((H // 2, H)), _full_spec((1, H // 2)),
        _full_spec((2, H // 2)), _full_spec((1, 2)),
    ],
    out_specs=[_row_spec(2)],
    out_shape=[jax.ShapeDtypeStruct((NP, 2), jnp.float32)],
)


def kernel(x, edge_index, Wl0, bl0, Wr0, g0, be0, Wl1, bl1, Wr1, g1, be1,
           Wl2, bl2, Wr2, g2, be2, Wc1, bc1, Wc2, bc2):
    src = edge_index[0]
    dst = edge_index[1]
    pad = EP - E
    src_p = jnp.concatenate([src, jnp.zeros((pad,), jnp.int32)])
    dst_p = jnp.concatenate([dst, jnp.full((pad,), TRASH, jnp.int32)])

    xp = jnp.pad(x, ((0, NP - N), (0, 0)))
    x_lo = xp[:, : D // 2]
    x_hi = xp[:, D // 2:]

    r = lambda v: v.reshape(1, -1)

    agg0_lo, agg0_hi, cnt = _sc_agg0(x_lo, x_hi, src_p, dst_p)
    h1_lo, h1_hi = _tc_layer0(agg0_lo, agg0_hi, cnt, x_lo, x_hi,
                              Wl0, r(bl0), Wr0, r(g0), r(be0))
    agg1_lo, agg1_hi = _sc_agg(h1_lo, h1_hi, src_p, dst_p)
    h2_lo, h2_hi = _tc_layer(agg1_lo, agg1_hi, cnt, h1_lo, h1_hi,
                             Wl1, r(bl1), Wr1, r(g1), r(be1))
    agg2_lo, agg2_hi = _sc_agg(h2_lo, h2_hi, src_p, dst_p)
    (out_p,) = _tc_head(agg2_lo, agg2_hi, cnt, h2_lo, h2_hi,
                        Wl2, r(bl2), Wr2, r(g2), r(be2),
                        Wc1, r(bc1), Wc2, r(bc2))
    return out_p[:N]
